# baseline (speedup 1.0000x reference)
import functools

import jax
import jax.numpy as jnp
from jax import lax
from jax.experimental import pallas as pl
from jax.experimental.pallas import tpu as pltpu
from jax.experimental.pallas import tpu_sc as plsc

N = 10000
E = 320000
G = 64
NPAD = 10240
NC, NS = 2, 16
NW = NC * NS
CH = 128
NCHT = 160
EPADT = NS * NCHT * CH
K0 = 48
KB_DEG = 8
RPT = NPAD // NS
FD = 16

_mesh = plsc.VectorSubcoreMesh(core_axis_name="c", subcore_axis_name="s")
_sc_params = pltpu.CompilerParams(use_tc_tiling_on_sc=False)


def _make_lap(F, KB):
    @functools.partial(
        pl.kernel,
        out_type=jax.ShapeDtypeStruct((NC, NPAD, F), jnp.float32),
        mesh=_mesh,
        scratch_types=[
            pltpu.VMEM_SHARED((NPAD, F), jnp.float32),
            pltpu.VMEM((NCHT, CH), jnp.int32),
            pltpu.VMEM((NCHT, CH), jnp.int32),
            pltpu.VMEM((KB, CH, F), jnp.float32),
            pltpu.SemaphoreType.DMA,
            pltpu.SemaphoreType.DMA,
        ],
        compiler_params=_sc_params,
    )
    def lap(src_hbm, dst_hbm, table_hbm, zeros_hbm, out_hbm,
            acc, src_v, dst_v, rows, gsem, ssem):
        c = lax.axis_index("c")
        s = lax.axis_index("s")
        row0 = s * RPT
        pltpu.sync_copy(zeros_hbm.at[pl.ds(row0, RPT)], acc.at[pl.ds(row0, RPT)])
        pltpu.sync_copy(src_hbm.at[s], src_v)
        pltpu.sync_copy(dst_hbm.at[s], dst_v)
        plsc.subcore_barrier()
        base = jnp.where(c == 0, 0, K0)
        ngroups = jnp.where(c == 0, K0 // KB, (NCHT - K0) // KB)

        def group(g, carry):
            j0 = base + g * KB
            for b in range(KB):
                pltpu.async_copy(table_hbm.at[src_v.at[j0 + b]], rows.at[b], gsem)
            for b in range(KB):
                pltpu.make_async_copy(
                    table_hbm.at[src_v.at[j0 + b]], rows.at[b], gsem).wait()
            for b in range(KB):
                pltpu.async_copy(rows.at[b], acc.at[dst_v.at[j0 + b]], ssem,
                                 add=True)
            for b in range(KB):
                pltpu.make_async_copy(
                    rows.at[b], acc.at[dst_v.at[j0 + b]], ssem).wait()
            return carry

        lax.fori_loop(0, ngroups, group, 0)
        plsc.subcore_barrier()
        pltpu.sync_copy(acc.at[pl.ds(row0, RPT)], out_hbm.at[c, pl.ds(row0, RPT)])

    return lap


_lap64 = _make_lap(64, 4)
_lap32 = _make_lap(32, 8)


@functools.partial(
    pl.kernel,
    out_type=jax.ShapeDtypeStruct((NC, NPAD, FD), jnp.float32),
    mesh=_mesh,
    scratch_types=[
        pltpu.VMEM_SHARED((NPAD, FD), jnp.float32),
        pltpu.VMEM((NCHT, CH), jnp.int32),
        pltpu.VMEM((CH, FD), jnp.float32),
        pltpu.SemaphoreType.DMA,
    ],
    compiler_params=_sc_params,
)
def _deg_sc(dst_hbm, zeros_hbm, ones_hbm, out_hbm, acc, dst_v, ones_v, ssem):
    c = lax.axis_index("c")
    s = lax.axis_index("s")
    row0 = s * RPT
    pltpu.sync_copy(zeros_hbm.at[pl.ds(row0, RPT)], acc.at[pl.ds(row0, RPT)])
    pltpu.sync_copy(ones_hbm, ones_v)
    pltpu.sync_copy(dst_hbm.at[s], dst_v)
    plsc.subcore_barrier()
    base = c * (NCHT // 2)

    def group(g, carry):
        j0 = base + g * KB_DEG
        for b in range(KB_DEG):
            pltpu.async_copy(ones_v, acc.at[dst_v.at[j0 + b]], ssem, add=True)
        for b in range(KB_DEG):
            pltpu.make_async_copy(ones_v, acc.at[dst_v.at[j0 + b]], ssem).wait()
        return carry

    lax.fori_loop(0, NCHT // 2 // KB_DEG, group, 0)
    plsc.subcore_barrier()
    pltpu.sync_copy(acc.at[pl.ds(row0, RPT)], out_hbm.at[c, pl.ds(row0, RPT)])


def _dot(a, b):
    return jnp.dot(a, b, preferred_element_type=jnp.float32)


def _tc1_body(p_ref, x_ref, w_ref, dis_ref, a_ref, u0_ref, u2_ref):
    deg = p_ref[0][:N, 0:1] + p_ref[1][:N, 0:1]
    dis = jnp.where(deg > 0, lax.rsqrt(jnp.maximum(deg, 1e-12)), 0.0)
    u = _dot(x_ref[...], w_ref[...])
    u1 = u[:, 32:64]
    u2 = u[:, 64:96]
    dis_ref[...] = dis
    a_ref[...] = jnp.concatenate([dis * u1, dis * u2], axis=1)
    u0_ref[...] = u[:, 0:32]
    u2_ref[...] = u2


def _tc2_body(p_ref, dis_ref, s1_ref, t3_ref):
    sp = p_ref[0][:N] + p_ref[1][:N]
    dis = dis_ref[...]
    s1_ref[...] = sp[:, 0:32]
    t3_ref[...] = (dis * dis) * sp[:, 32:64]


def _tc3_body(u0_ref, u2_ref, s1_ref, q_ref, dis_ref, b_ref,
              w0_ref, w1_ref, w2_ref, a_ref, v0_ref, v2_ref):
    dis = dis_ref[...]
    s3 = q_ref[0][:N] + q_ref[1][:N]
    h = jax.nn.relu(u0_ref[...] - dis * s1_ref[...] + 2.0 * dis * s3
                    - u2_ref[...] + b_ref[...])
    v1 = _dot(h, w1_ref[...])
    v2 = _dot(h, w2_ref[...])
    a_ref[...] = jnp.concatenate([dis * v1, dis * v2], axis=1)
    v0_ref[...] = _dot(h, w0_ref[...])
    v2_ref[...] = v2


def _tc5_body(v0_ref, v2_ref, s4_ref, q_ref, dis_ref, b_ref, fcw_ref,
              fcb_ref, batch_ref, out_ref):
    dis = dis_ref[...]
    s6 = q_ref[0][:N] + q_ref[1][:N]
    h = jax.nn.relu(v0_ref[...] - dis * s4_ref[...] + 2.0 * dis * s6
                    - v2_ref[...] + b_ref[...])
    r = _dot(h, fcw_ref[...])
    gid = lax.broadcasted_iota(jnp.int32, (G, N), 0)
    m = (batch_ref[...] == gid).astype(jnp.float32)
    pooled = _dot(m, r)
    cnt = jnp.sum(m, axis=1, keepdims=True)
    out_ref[...] = pooled / jnp.maximum(cnt, 1.0) + fcb_ref[...]


def _f32(shape):
    return jax.ShapeDtypeStruct(shape, jnp.float32)


_tc1 = pl.pallas_call(
    _tc1_body, out_shape=(_f32((N, 1)), _f32((N, 64)), _f32((N, 32)), _f32((N, 32))))
_tc2 = pl.pallas_call(_tc2_body, out_shape=(_f32((N, 32)), _f32((N, 32))))
_tc3 = pl.pallas_call(
    _tc3_body, out_shape=(_f32((N, 64)), _f32((N, 32)), _f32((N, 32))))
_tc5 = pl.pallas_call(_tc5_body, out_shape=_f32((G, 1)))


def kernel(x, edge_index, batch, W1, b1, W2, b2, fc_w, fc_b):
    npad_e = EPADT - E
    src = jnp.concatenate(
        [edge_index[0], jnp.zeros((npad_e,), jnp.int32)]).reshape(NS, NCHT, CH)
    dst = jnp.concatenate(
        [edge_index[1], jnp.full((npad_e,), NPAD - 1, jnp.int32)]
    ).reshape(NS, NCHT, CH)
    w1all = jnp.concatenate([W1[0], W1[1], W1[2]], axis=1)
    z64 = jnp.zeros((NPAD, 64), jnp.float32)
    z32 = jnp.zeros((NPAD, 32), jnp.float32)
    z16 = jnp.zeros((NPAD, FD), jnp.float32)
    ones16 = jnp.ones((CH, FD), jnp.float32)

    degp = _deg_sc(dst, z16, ones16)
    dis, a, u0, u2 = _tc1(degp, x, w1all)
    p1 = _lap64(src, dst, a, z64)
    s1, t3 = _tc2(p1, dis)
    q1 = _lap32(src, dst, t3, z32)
    bt, v0, v2 = _tc3(u0, u2, s1, q1, dis, b1.reshape(1, 32),
                      W2[0], W2[1], W2[2])
    p2 = _lap64(src, dst, bt, z64)
    s4, t6 = _tc2(p2, dis)
    q2 = _lap32(src, dst, t6, z32)
    out = _tc5(v0, v2, s4, q2, dis, b2.reshape(1, 32), fc_w,
               fc_b.reshape(1, 1), batch.reshape(1, N))
    return out.reshape(G)


# speedup vs baseline: 1.2672x; 1.2672x over previous
import functools

import jax
import jax.numpy as jnp
from jax import lax
from jax.experimental import pallas as pl
from jax.experimental.pallas import tpu as pltpu
from jax.experimental.pallas import tpu_sc as plsc

N = 10000
E = 320000
G = 64
NPAD = 10240
NC, NS = 2, 16
NW = NC * NS
CH = 128
NCHT = 160
EPADT = NS * NCHT * CH
K0 = 128
KB_DEG = 8
RPT = NPAD // NS
FD = 16

_mesh = plsc.VectorSubcoreMesh(core_axis_name="c", subcore_axis_name="s")
_sc_params = pltpu.CompilerParams(use_tc_tiling_on_sc=False)


def _make_lap(F, KB):
    @functools.partial(
        pl.kernel,
        out_type=jax.ShapeDtypeStruct((NC, NPAD, F), jnp.float32),
        mesh=_mesh,
        scratch_types=[
            pltpu.VMEM_SHARED((NPAD, F), jnp.float32),
            pltpu.VMEM((NCHT, CH), jnp.int32),
            pltpu.VMEM((NCHT, CH), jnp.int32),
            pltpu.VMEM((KB, CH, F), jnp.float32),
            pltpu.SemaphoreType.DMA,
            pltpu.SemaphoreType.DMA,
        ],
        compiler_params=_sc_params,
    )
    def lap(src_hbm, dst_hbm, table_hbm, zeros_hbm, out_hbm,
            acc, src_v, dst_v, rows, gsem, ssem):
        c = lax.axis_index("c")
        s = lax.axis_index("s")
        row0 = s * RPT
        pltpu.sync_copy(zeros_hbm.at[pl.ds(row0, RPT)], acc.at[pl.ds(row0, RPT)])
        pltpu.sync_copy(src_hbm.at[s], src_v)
        pltpu.sync_copy(dst_hbm.at[s], dst_v)
        plsc.subcore_barrier()
        base = jnp.where(c == 0, 0, K0)
        ngroups = jnp.where(c == 0, K0 // KB, (NCHT - K0) // KB)

        def group(g, carry):
            j0 = base + g * KB
            for b in range(KB):
                pltpu.async_copy(table_hbm.at[src_v.at[j0 + b]], rows.at[b], gsem)
            for b in range(KB):
                pltpu.make_async_copy(
                    table_hbm.at[src_v.at[j0 + b]], rows.at[b], gsem).wait()
            for b in range(KB):
                pltpu.async_copy(rows.at[b], acc.at[dst_v.at[j0 + b]], ssem,
                                 add=True)
            for b in range(KB):
                pltpu.make_async_copy(
                    rows.at[b], acc.at[dst_v.at[j0 + b]], ssem).wait()
            return carry

        lax.fori_loop(0, ngroups, group, 0)
        plsc.subcore_barrier()
        pltpu.sync_copy(acc.at[pl.ds(row0, RPT)], out_hbm.at[c, pl.ds(row0, RPT)])

    return lap


_lap64 = _make_lap(64, 4)
_lap32 = _make_lap(32, 8)


@functools.partial(
    pl.kernel,
    out_type=jax.ShapeDtypeStruct((NC, NPAD, FD), jnp.float32),
    mesh=_mesh,
    scratch_types=[
        pltpu.VMEM_SHARED((NPAD, FD), jnp.float32),
        pltpu.VMEM((NCHT, CH), jnp.int32),
        pltpu.VMEM((CH, FD), jnp.float32),
        pltpu.SemaphoreType.DMA,
    ],
    compiler_params=_sc_params,
)
def _deg_sc(dst_hbm, zeros_hbm, ones_hbm, out_hbm, acc, dst_v, ones_v, ssem):
    c = lax.axis_index("c")
    s = lax.axis_index("s")
    row0 = s * RPT
    pltpu.sync_copy(zeros_hbm.at[pl.ds(row0, RPT)], acc.at[pl.ds(row0, RPT)])
    pltpu.sync_copy(ones_hbm, ones_v)
    pltpu.sync_copy(dst_hbm.at[s], dst_v)
    plsc.subcore_barrier()
    base = c * (NCHT // 2)

    def group(g, carry):
        j0 = base + g * KB_DEG
        for b in range(KB_DEG):
            pltpu.async_copy(ones_v, acc.at[dst_v.at[j0 + b]], ssem, add=True)
        for b in range(KB_DEG):
            pltpu.make_async_copy(ones_v, acc.at[dst_v.at[j0 + b]], ssem).wait()
        return carry

    lax.fori_loop(0, NCHT // 2 // KB_DEG, group, 0)
    plsc.subcore_barrier()
    pltpu.sync_copy(acc.at[pl.ds(row0, RPT)], out_hbm.at[c, pl.ds(row0, RPT)])


def _dot(a, b):
    return jnp.dot(a, b, preferred_element_type=jnp.float32)


def _tc1_body(p_ref, x_ref, w_ref, dis_ref, a_ref, u0_ref, u2_ref):
    deg = p_ref[0][:N, 0:1] + p_ref[1][:N, 0:1]
    dis = jnp.where(deg > 0, lax.rsqrt(jnp.maximum(deg, 1e-12)), 0.0)
    u = _dot(x_ref[...], w_ref[...])
    u1 = u[:, 32:64]
    u2 = u[:, 64:96]
    dis_ref[...] = dis
    a_ref[...] = jnp.concatenate([dis * u1, dis * u2], axis=1)
    u0_ref[...] = u[:, 0:32]
    u2_ref[...] = u2


def _tc2_body(p_ref, dis_ref, s1_ref, t3_ref):
    sp = p_ref[0][:N] + p_ref[1][:N]
    dis = dis_ref[...]
    s1_ref[...] = sp[:, 0:32]
    t3_ref[...] = (dis * dis) * sp[:, 32:64]


def _tc3_body(u0_ref, u2_ref, s1_ref, q_ref, dis_ref, b_ref,
              w0_ref, w1_ref, w2_ref, a_ref, v0_ref, v2_ref):
    dis = dis_ref[...]
    s3 = q_ref[0][:N] + q_ref[1][:N]
    h = jax.nn.relu(u0_ref[...] - dis * s1_ref[...] + 2.0 * dis * s3
                    - u2_ref[...] + b_ref[...])
    v1 = _dot(h, w1_ref[...])
    v2 = _dot(h, w2_ref[...])
    a_ref[...] = jnp.concatenate([dis * v1, dis * v2], axis=1)
    v0_ref[...] = _dot(h, w0_ref[...])
    v2_ref[...] = v2


def _tc5_body(v0_ref, v2_ref, s4_ref, q_ref, dis_ref, b_ref, fcw_ref,
              fcb_ref, batch_ref, out_ref):
    dis = dis_ref[...]
    s6 = q_ref[0][:N] + q_ref[1][:N]
    h = jax.nn.relu(v0_ref[...] - dis * s4_ref[...] + 2.0 * dis * s6
                    - v2_ref[...] + b_ref[...])
    r = _dot(h, fcw_ref[...])
    gid = lax.broadcasted_iota(jnp.int32, (G, N), 0)
    m = (batch_ref[...] == gid).astype(jnp.float32)
    pooled = _dot(m, r)
    cnt = jnp.sum(m, axis=1, keepdims=True)
    out_ref[...] = pooled / jnp.maximum(cnt, 1.0) + fcb_ref[...]


def _f32(shape):
    return jax.ShapeDtypeStruct(shape, jnp.float32)


_tc1 = pl.pallas_call(
    _tc1_body, out_shape=(_f32((N, 1)), _f32((N, 64)), _f32((N, 32)), _f32((N, 32))))
_tc2 = pl.pallas_call(_tc2_body, out_shape=(_f32((N, 32)), _f32((N, 32))))
_tc3 = pl.pallas_call(
    _tc3_body, out_shape=(_f32((N, 64)), _f32((N, 32)), _f32((N, 32))))
_tc5 = pl.pallas_call(_tc5_body, out_shape=_f32((G, 1)))


def kernel(x, edge_index, batch, W1, b1, W2, b2, fc_w, fc_b):
    npad_e = EPADT - E
    src = jnp.concatenate(
        [edge_index[0], jnp.zeros((npad_e,), jnp.int32)]).reshape(NS, NCHT, CH)
    dst = jnp.concatenate(
        [edge_index[1], jnp.full((npad_e,), NPAD - 1, jnp.int32)]
    ).reshape(NS, NCHT, CH)
    w1all = jnp.concatenate([W1[0], W1[1], W1[2]], axis=1)
    z64 = jnp.zeros((NPAD, 64), jnp.float32)
    z32 = jnp.zeros((NPAD, 32), jnp.float32)
    z16 = jnp.zeros((NPAD, FD), jnp.float32)
    ones16 = jnp.ones((CH, FD), jnp.float32)

    degp = _deg_sc(dst, z16, ones16)
    dis, a, u0, u2 = _tc1(degp, x, w1all)
    p1 = _lap64(src, dst, a, z64)
    s1, t3 = _tc2(p1, dis)
    q1 = _lap32(src, dst, t3, z32)
    bt, v0, v2 = _tc3(u0, u2, s1, q1, dis, b1.reshape(1, 32),
                      W2[0], W2[1], W2[2])
    p2 = _lap64(src, dst, bt, z64)
    s4, t6 = _tc2(p2, dis)
    q2 = _lap32(src, dst, t6, z32)
    out = _tc5(v0, v2, s4, q2, dis, b2.reshape(1, 32), fc_w,
               fc_b.reshape(1, 1), batch.reshape(1, N))
    return out.reshape(G)


# speedup vs baseline: 2.3484x; 1.8533x over previous
import functools

import jax
import jax.numpy as jnp
from jax import lax
from jax.experimental import pallas as pl
from jax.experimental.pallas import tpu as pltpu
from jax.experimental.pallas import tpu_sc as plsc

N = 10000
E = 320000
G = 64
NPAD = 10240
NC, NS = 2, 16
CH = 128
NCHT = 160
EPADT = NS * NCHT * CH
K0 = 80
KB = 8
KB_DEG = 8
RPT = NPAD // NS
SPT = NPAD // NS
NB = 1024
GRID = NPAD // NB
FD = 16
F = 32

_mesh = plsc.VectorSubcoreMesh(core_axis_name="c", subcore_axis_name="s")
_sc_params = pltpu.CompilerParams(use_tc_tiling_on_sc=False)


@functools.partial(
    pl.kernel,
    out_type=jax.ShapeDtypeStruct((NC, NPAD, F), jnp.float32),
    mesh=_mesh,
    scratch_types=[
        pltpu.VMEM_SHARED((NPAD, F), jnp.float32),
        pltpu.VMEM_SHARED((NPAD, F), jnp.float32),
        pltpu.VMEM((NCHT, CH), jnp.int32),
        pltpu.VMEM((NCHT, CH), jnp.int32),
        pltpu.VMEM((KB, CH, F), jnp.float32),
        pltpu.SemaphoreType.DMA,
        pltpu.SemaphoreType.DMA,
    ],
    compiler_params=_sc_params,
)
def _lap_sc(src_hbm, dst_hbm, table_hbm, zeros_hbm, out_hbm,
            table_s, acc, src_v, dst_v, rows, gsem, ssem):
    c = lax.axis_index("c")
    s = lax.axis_index("s")
    row0 = s * RPT
    trow0 = s * SPT
    pltpu.sync_copy(zeros_hbm.at[pl.ds(row0, RPT)], acc.at[pl.ds(row0, RPT)])
    pltpu.sync_copy(table_hbm.at[pl.ds(trow0, SPT)],
                    table_s.at[pl.ds(trow0, SPT)])
    pltpu.sync_copy(src_hbm.at[s], src_v)
    pltpu.sync_copy(dst_hbm.at[s], dst_v)
    plsc.subcore_barrier()
    base = jnp.where(c == 0, 0, K0)
    ngroups = jnp.where(c == 0, K0 // KB, (NCHT - K0) // KB)

    def group(g, carry):
        j0 = base + g * KB
        for b in range(KB):
            pltpu.async_copy(table_s.at[src_v.at[j0 + b]], rows.at[b], gsem)
        for b in range(KB):
            pltpu.make_async_copy(
                table_s.at[src_v.at[j0 + b]], rows.at[b], gsem).wait()
        for b in range(KB):
            pltpu.async_copy(rows.at[b], acc.at[dst_v.at[j0 + b]], ssem,
                             add=True)
        for b in range(KB):
            pltpu.make_async_copy(
                rows.at[b], acc.at[dst_v.at[j0 + b]], ssem).wait()
        return carry

    lax.fori_loop(0, ngroups, group, 0)
    plsc.subcore_barrier()
    pltpu.sync_copy(acc.at[pl.ds(row0, RPT)], out_hbm.at[c, pl.ds(row0, RPT)])


@functools.partial(
    pl.kernel,
    out_type=jax.ShapeDtypeStruct((NC, NPAD, FD), jnp.float32),
    mesh=_mesh,
    scratch_types=[
        pltpu.VMEM_SHARED((NPAD, FD), jnp.float32),
        pltpu.VMEM((NCHT, CH), jnp.int32),
        pltpu.VMEM((CH, FD), jnp.float32),
        pltpu.SemaphoreType.DMA,
    ],
    compiler_params=_sc_params,
)
def _deg_sc(dst_hbm, zeros_hbm, ones_hbm, out_hbm, acc, dst_v, ones_v, ssem):
    c = lax.axis_index("c")
    s = lax.axis_index("s")
    row0 = s * RPT
    pltpu.sync_copy(zeros_hbm.at[pl.ds(row0, RPT)], acc.at[pl.ds(row0, RPT)])
    pltpu.sync_copy(ones_hbm, ones_v)
    pltpu.sync_copy(dst_hbm.at[s], dst_v)
    plsc.subcore_barrier()
    base = c * (NCHT // 2)

    def group(g, carry):
        j0 = base + g * KB_DEG
        for b in range(KB_DEG):
            pltpu.async_copy(ones_v, acc.at[dst_v.at[j0 + b]], ssem, add=True)
        for b in range(KB_DEG):
            pltpu.make_async_copy(ones_v, acc.at[dst_v.at[j0 + b]], ssem).wait()
        return carry

    lax.fori_loop(0, NCHT // 2 // KB_DEG, group, 0)
    plsc.subcore_barrier()
    pltpu.sync_copy(acc.at[pl.ds(row0, RPT)], out_hbm.at[c, pl.ds(row0, RPT)])


def _dot(a, b):
    return jnp.dot(a, b, preferred_element_type=jnp.float32)


def _tc1_body(p_ref, x_ref, w_ref, dis_ref, t1_ref, t2_ref, u0_ref, u2_ref):
    deg = p_ref[0][:, 0:1] + p_ref[1][:, 0:1]
    dis = jnp.where(deg > 0, lax.rsqrt(jnp.maximum(deg, 1e-12)), 0.0)
    u = _dot(x_ref[...], w_ref[...])
    u2 = u[:, 64:96]
    dis_ref[...] = dis
    t1_ref[...] = dis * u[:, 32:64]
    t2_ref[...] = dis * u2
    u0_ref[...] = u[:, 0:32]
    u2_ref[...] = u2


def _tc2_body(q_ref, dis_ref, t3_ref):
    dis = dis_ref[...]
    t3_ref[...] = (dis * dis) * (q_ref[0] + q_ref[1])


def _tc3_body(u0_ref, u2_ref, p_ref, q_ref, dis_ref, b_ref,
              w0_ref, w1_ref, w2_ref, t4_ref, t5_ref, v0_ref, v2_ref):
    dis = dis_ref[...]
    s1 = p_ref[0] + p_ref[1]
    s3 = q_ref[0] + q_ref[1]
    h = jax.nn.relu(u0_ref[...] - dis * s1 + 2.0 * dis * s3
                    - u2_ref[...] + b_ref[...])
    v2 = _dot(h, w2_ref[...])
    t4_ref[...] = dis * _dot(h, w1_ref[...])
    t5_ref[...] = dis * v2
    v0_ref[...] = _dot(h, w0_ref[...])
    v2_ref[...] = v2


def _tc5_body(v0_ref, v2_ref, p_ref, q_ref, dis_ref, b_ref, fcw_ref,
              batch_ref, psum_ref, cnt_ref):
    i = pl.program_id(0)
    dis = dis_ref[...]
    s4 = p_ref[0] + p_ref[1]
    s6 = q_ref[0] + q_ref[1]
    h = jax.nn.relu(v0_ref[...] - dis * s4 + 2.0 * dis * s6
                    - v2_ref[...] + b_ref[...])
    r = _dot(h, fcw_ref[...])
    gid = lax.broadcasted_iota(jnp.int32, (G, NB), 0)
    m = (batch_ref[...] == gid).astype(jnp.float32)
    ps = _dot(m, r)
    ct = jnp.sum(m, axis=1, keepdims=True)

    @pl.when(i == 0)
    def _():
        psum_ref[...] = ps
        cnt_ref[...] = ct

    @pl.when(i > 0)
    def _():
        psum_ref[...] = psum_ref[...] + ps
        cnt_ref[...] = cnt_ref[...] + ct


def _tc6_body(psum_ref, cnt_ref, fcb_ref, out_ref):
    out_ref[...] = (psum_ref[...] / jnp.maximum(cnt_ref[...], 1.0)
                    + fcb_ref[...])


def _f32(shape):
    return jax.ShapeDtypeStruct(shape, jnp.float32)


def _row(width):
    return pl.BlockSpec((NB, width), lambda i: (i, 0))


def _prt(width):
    return pl.BlockSpec((2, NB, width), lambda i: (0, i, 0))


def _full(shape):
    return pl.BlockSpec(shape, lambda i: tuple(0 for _ in shape))


_tc1 = pl.pallas_call(
    _tc1_body, grid=(GRID,),
    in_specs=[_prt(FD), _row(128), _full((128, 96))],
    out_specs=(_row(1), _row(32), _row(32), _row(32), _row(32)),
    out_shape=(_f32((NPAD, 1)), _f32((NPAD, 32)), _f32((NPAD, 32)),
               _f32((NPAD, 32)), _f32((NPAD, 32))))
_tc2 = pl.pallas_call(
    _tc2_body, grid=(GRID,),
    in_specs=[_prt(32), _row(1)],
    out_specs=_row(32), out_shape=_f32((NPAD, 32)))
_tc3 = pl.pallas_call(
    _tc3_body, grid=(GRID,),
    in_specs=[_row(32), _row(32), _prt(32), _prt(32), _row(1),
              _full((1, 32)), _full((32, 32)), _full((32, 32)),
              _full((32, 32))],
    out_specs=(_row(32), _row(32), _row(32), _row(32)),
    out_shape=(_f32((NPAD, 32)), _f32((NPAD, 32)), _f32((NPAD, 32)),
               _f32((NPAD, 32))))
_tc5 = pl.pallas_call(
    _tc5_body, grid=(GRID,),
    in_specs=[_row(32), _row(32), _prt(32), _prt(32), _row(1),
              _full((1, 32)), _full((32, 1)),
              pl.BlockSpec((1, NB), lambda i: (0, i))],
    out_specs=(_full((G, 1)), _full((G, 1))),
    out_shape=(_f32((G, 1)), _f32((G, 1))))
_tc6 = pl.pallas_call(
    _tc6_body, out_shape=_f32((G, 1)))


def kernel(x, edge_index, batch, W1, b1, W2, b2, fc_w, fc_b):
    npad_e = EPADT - E
    src = jnp.concatenate(
        [edge_index[0], jnp.zeros((npad_e,), jnp.int32)]).reshape(NS, NCHT, CH)
    dst = jnp.concatenate(
        [edge_index[1], jnp.full((npad_e,), NPAD - 1, jnp.int32)]
    ).reshape(NS, NCHT, CH)
    xp = jnp.concatenate([x, jnp.zeros((NPAD - N, 128), jnp.float32)])
    bp = jnp.concatenate(
        [batch, jnp.full((NPAD - N,), G, jnp.int32)]).reshape(1, NPAD)
    w1all = jnp.concatenate([W1[0], W1[1], W1[2]], axis=1)
    z32 = jnp.zeros((NPAD, F), jnp.float32)
    z16 = jnp.zeros((NPAD, FD), jnp.float32)
    ones16 = jnp.ones((CH, FD), jnp.float32)

    degp = _deg_sc(dst, z16, ones16)
    dis, t1, t2, u0, u2 = _tc1(degp, xp, w1all)
    ps1 = _lap_sc(src, dst, t1, z32)
    ps2 = _lap_sc(src, dst, t2, z32)
    t3 = _tc2(ps2, dis)
    ps3 = _lap_sc(src, dst, t3, z32)
    t4, t5, v0, v2 = _tc3(u0, u2, ps1, ps3, dis, b1.reshape(1, 32),
                          W2[0], W2[1], W2[2])
    ps4 = _lap_sc(src, dst, t4, z32)
    ps5 = _lap_sc(src, dst, t5, z32)
    t6 = _tc2(ps5, dis)
    ps6 = _lap_sc(src, dst, t6, z32)
    psum, cnt = _tc5(v0, v2, ps4, ps6, dis, b2.reshape(1, 32), fc_w, bp)
    out = _tc6(psum, cnt, fc_b.reshape(1, 1))
    return out.reshape(G)


# speedup vs baseline: 2.5936x; 1.1044x over previous
import functools

import jax
import jax.numpy as jnp
from jax import lax
from jax.experimental import pallas as pl
from jax.experimental.pallas import tpu as pltpu
from jax.experimental.pallas import tpu_sc as plsc

N = 10000
E = 320000
G = 64
NPAD = 10240
NC, NS = 2, 16
CH = 128
NCHT = 160
EPADT = NS * NCHT * CH
K0 = 80
KB = 4
KB_DEG = 8
RPT = NPAD // NS
SPT = NPAD // NS
NB = 1024
GRID = NPAD // NB
FD = 16
F = 32

_mesh = plsc.VectorSubcoreMesh(core_axis_name="c", subcore_axis_name="s")
_sc_params = pltpu.CompilerParams(use_tc_tiling_on_sc=False)


@functools.partial(
    pl.kernel,
    out_type=jax.ShapeDtypeStruct((NC, NPAD, F), jnp.float32),
    mesh=_mesh,
    scratch_types=[
        pltpu.VMEM_SHARED((NPAD, F), jnp.float32),
        pltpu.VMEM_SHARED((NPAD, F), jnp.float32),
        pltpu.VMEM((NCHT, CH), jnp.int32),
        pltpu.VMEM((NCHT, CH), jnp.int32),
        pltpu.VMEM((2, KB, CH, F), jnp.float32),
        pltpu.SemaphoreType.DMA,
        pltpu.SemaphoreType.DMA,
        pltpu.SemaphoreType.DMA,
        pltpu.SemaphoreType.DMA,
    ],
    compiler_params=_sc_params,
)
def _lap_sc(src_hbm, dst_hbm, table_hbm, zeros_hbm, out_hbm,
            table_s, acc, src_v, dst_v, rows, gsem0, gsem1, ssem0, ssem1):
    c = lax.axis_index("c")
    s = lax.axis_index("s")
    row0 = s * RPT
    pltpu.sync_copy(zeros_hbm.at[pl.ds(row0, RPT)], acc.at[pl.ds(row0, RPT)])
    pltpu.sync_copy(table_hbm.at[pl.ds(row0, SPT)],
                    table_s.at[pl.ds(row0, SPT)])
    pltpu.sync_copy(src_hbm.at[s], src_v)
    pltpu.sync_copy(dst_hbm.at[s], dst_v)
    plsc.subcore_barrier()
    base = jnp.where(c == 0, 0, K0)
    count = jnp.where(c == 0, K0, NCHT - K0)
    npairs = count // (2 * KB)
    last = base + count - KB

    def fire_g(st, j0, sem):
        for b in range(KB):
            pltpu.async_copy(table_s.at[src_v.at[j0 + b]], rows.at[st, b], sem)

    def drain_g(st, j0, sem):
        for b in range(KB):
            pltpu.make_async_copy(
                table_s.at[src_v.at[j0 + b]], rows.at[st, b], sem).wait()

    def fire_s(st, j0, sem):
        for b in range(KB):
            pltpu.async_copy(rows.at[st, b], acc.at[dst_v.at[j0 + b]], sem,
                             add=True)

    def drain_s(st, j0, sem):
        for b in range(KB):
            pltpu.make_async_copy(
                rows.at[st, b], acc.at[dst_v.at[j0 + b]], sem).wait()

    fire_g(0, base, gsem0)

    def pair(t, carry):
        ja = base + (2 * t) * KB
        jb = base + (2 * t + 1) * KB
        jn = jnp.minimum(ja + 2 * KB, last)
        drain_g(0, ja, gsem0)
        fire_s(0, ja, ssem0)
        fire_g(1, jb, gsem1)
        drain_s(0, ja, ssem0)
        drain_g(1, jb, gsem1)
        fire_s(1, jb, ssem1)
        fire_g(0, jn, gsem0)
        drain_s(1, jb, ssem1)
        return carry

    lax.fori_loop(0, npairs, pair, 0)
    drain_g(0, last, gsem0)
    plsc.subcore_barrier()
    pltpu.sync_copy(acc.at[pl.ds(row0, RPT)], out_hbm.at[c, pl.ds(row0, RPT)])


@functools.partial(
    pl.kernel,
    out_type=jax.ShapeDtypeStruct((NC, NPAD, FD), jnp.float32),
    mesh=_mesh,
    scratch_types=[
        pltpu.VMEM_SHARED((NPAD, FD), jnp.float32),
        pltpu.VMEM((NCHT, CH), jnp.int32),
        pltpu.VMEM((CH, FD), jnp.float32),
        pltpu.SemaphoreType.DMA,
    ],
    compiler_params=_sc_params,
)
def _deg_sc(dst_hbm, zeros_hbm, ones_hbm, out_hbm, acc, dst_v, ones_v, ssem):
    c = lax.axis_index("c")
    s = lax.axis_index("s")
    row0 = s * RPT
    pltpu.sync_copy(zeros_hbm.at[pl.ds(row0, RPT)], acc.at[pl.ds(row0, RPT)])
    pltpu.sync_copy(ones_hbm, ones_v)
    pltpu.sync_copy(dst_hbm.at[s], dst_v)
    plsc.subcore_barrier()
    base = c * (NCHT // 2)

    def group(g, carry):
        j0 = base + g * KB_DEG
        for b in range(KB_DEG):
            pltpu.async_copy(ones_v, acc.at[dst_v.at[j0 + b]], ssem, add=True)
        for b in range(KB_DEG):
            pltpu.make_async_copy(ones_v, acc.at[dst_v.at[j0 + b]], ssem).wait()
        return carry

    lax.fori_loop(0, NCHT // 2 // KB_DEG, group, 0)
    plsc.subcore_barrier()
    pltpu.sync_copy(acc.at[pl.ds(row0, RPT)], out_hbm.at[c, pl.ds(row0, RPT)])


def _dot(a, b):
    return jnp.dot(a, b, preferred_element_type=jnp.float32)


def _tc1_body(p_ref, x_ref, w_ref, dis_ref, t1_ref, t2_ref, u0_ref, u2_ref):
    deg = p_ref[0][:, 0:1] + p_ref[1][:, 0:1]
    dis = jnp.where(deg > 0, lax.rsqrt(jnp.maximum(deg, 1e-12)), 0.0)
    u = _dot(x_ref[...], w_ref[...])
    u2 = u[:, 64:96]
    dis_ref[...] = dis
    t1_ref[...] = dis * u[:, 32:64]
    t2_ref[...] = dis * u2
    u0_ref[...] = u[:, 0:32]
    u2_ref[...] = u2


def _tc2_body(q_ref, dis_ref, t3_ref):
    dis = dis_ref[...]
    t3_ref[...] = (dis * dis) * (q_ref[0] + q_ref[1])


def _tc3_body(u0_ref, u2_ref, p_ref, q_ref, dis_ref, b_ref,
              w0_ref, w1_ref, w2_ref, t4_ref, t5_ref, v0_ref, v2_ref):
    dis = dis_ref[...]
    s1 = p_ref[0] + p_ref[1]
    s3 = q_ref[0] + q_ref[1]
    h = jax.nn.relu(u0_ref[...] - dis * s1 + 2.0 * dis * s3
                    - u2_ref[...] + b_ref[...])
    v2 = _dot(h, w2_ref[...])
    t4_ref[...] = dis * _dot(h, w1_ref[...])
    t5_ref[...] = dis * v2
    v0_ref[...] = _dot(h, w0_ref[...])
    v2_ref[...] = v2


def _tc5_body(v0_ref, v2_ref, p_ref, q_ref, dis_ref, b_ref, fcw_ref,
              batch_ref, psum_ref, cnt_ref):
    i = pl.program_id(0)
    dis = dis_ref[...]
    s4 = p_ref[0] + p_ref[1]
    s6 = q_ref[0] + q_ref[1]
    h = jax.nn.relu(v0_ref[...] - dis * s4 + 2.0 * dis * s6
                    - v2_ref[...] + b_ref[...])
    r = _dot(h, fcw_ref[...])
    gid = lax.broadcasted_iota(jnp.int32, (G, NB), 0)
    m = (batch_ref[...] == gid).astype(jnp.float32)
    ps = _dot(m, r)
    ct = jnp.sum(m, axis=1, keepdims=True)

    @pl.when(i == 0)
    def _():
        psum_ref[...] = ps
        cnt_ref[...] = ct

    @pl.when(i > 0)
    def _():
        psum_ref[...] = psum_ref[...] + ps
        cnt_ref[...] = cnt_ref[...] + ct


def _tc6_body(psum_ref, cnt_ref, fcb_ref, out_ref):
    out_ref[...] = (psum_ref[...] / jnp.maximum(cnt_ref[...], 1.0)
                    + fcb_ref[...])


def _f32(shape):
    return jax.ShapeDtypeStruct(shape, jnp.float32)


def _row(width):
    return pl.BlockSpec((NB, width), lambda i: (i, 0))


def _prt(width):
    return pl.BlockSpec((2, NB, width), lambda i: (0, i, 0))


def _full(shape):
    return pl.BlockSpec(shape, lambda i: tuple(0 for _ in shape))


_tc1 = pl.pallas_call(
    _tc1_body, grid=(GRID,),
    in_specs=[_prt(FD), _row(128), _full((128, 96))],
    out_specs=(_row(1), _row(32), _row(32), _row(32), _row(32)),
    out_shape=(_f32((NPAD, 1)), _f32((NPAD, 32)), _f32((NPAD, 32)),
               _f32((NPAD, 32)), _f32((NPAD, 32))))
_tc2 = pl.pallas_call(
    _tc2_body, grid=(GRID,),
    in_specs=[_prt(32), _row(1)],
    out_specs=_row(32), out_shape=_f32((NPAD, 32)))
_tc3 = pl.pallas_call(
    _tc3_body, grid=(GRID,),
    in_specs=[_row(32), _row(32), _prt(32), _prt(32), _row(1),
              _full((1, 32)), _full((32, 32)), _full((32, 32)),
              _full((32, 32))],
    out_specs=(_row(32), _row(32), _row(32), _row(32)),
    out_shape=(_f32((NPAD, 32)), _f32((NPAD, 32)), _f32((NPAD, 32)),
               _f32((NPAD, 32))))
_tc5 = pl.pallas_call(
    _tc5_body, grid=(GRID,),
    in_specs=[_row(32), _row(32), _prt(32), _prt(32), _row(1),
              _full((1, 32)), _full((32, 1)),
              pl.BlockSpec((1, NB), lambda i: (0, i))],
    out_specs=(_full((G, 1)), _full((G, 1))),
    out_shape=(_f32((G, 1)), _f32((G, 1))))
_tc6 = pl.pallas_call(
    _tc6_body, out_shape=_f32((G, 1)))


def kernel(x, edge_index, batch, W1, b1, W2, b2, fc_w, fc_b):
    npad_e = EPADT - E
    src = jnp.concatenate(
        [edge_index[0], jnp.zeros((npad_e,), jnp.int32)]).reshape(NS, NCHT, CH)
    dst = jnp.concatenate(
        [edge_index[1], jnp.full((npad_e,), NPAD - 1, jnp.int32)]
    ).reshape(NS, NCHT, CH)
    xp = jnp.concatenate([x, jnp.zeros((NPAD - N, 128), jnp.float32)])
    bp = jnp.concatenate(
        [batch, jnp.full((NPAD - N,), G, jnp.int32)]).reshape(1, NPAD)
    w1all = jnp.concatenate([W1[0], W1[1], W1[2]], axis=1)
    z32 = jnp.zeros((NPAD, F), jnp.float32)
    z16 = jnp.zeros((NPAD, FD), jnp.float32)
    ones16 = jnp.ones((CH, FD), jnp.float32)

    degp = _deg_sc(dst, z16, ones16)
    dis, t1, t2, u0, u2 = _tc1(degp, xp, w1all)
    ps1 = _lap_sc(src, dst, t1, z32)
    ps2 = _lap_sc(src, dst, t2, z32)
    t3 = _tc2(ps2, dis)
    ps3 = _lap_sc(src, dst, t3, z32)
    t4, t5, v0, v2 = _tc3(u0, u2, ps1, ps3, dis, b1.reshape(1, 32),
                          W2[0], W2[1], W2[2])
    ps4 = _lap_sc(src, dst, t4, z32)
    ps5 = _lap_sc(src, dst, t5, z32)
    t6 = _tc2(ps5, dis)
    ps6 = _lap_sc(src, dst, t6, z32)
    psum, cnt = _tc5(v0, v2, ps4, ps6, dis, b2.reshape(1, 32), fc_w, bp)
    out = _tc6(psum, cnt, fc_b.reshape(1, 1))
    return out.reshape(G)
